# baseline (speedup 1.0000x reference)
import functools

import jax
import jax.numpy as jnp
from jax import lax
from jax.experimental import pallas as pl
from jax.experimental.pallas import tpu as pltpu
from jax.experimental.pallas import tpu_sc as plsc

_C = 3
_IMG = 64
_H = 512
_W = 512
_NW = 32
_IPW = _IMG // _NW
_CR = 64
_CPI = _H // _CR
_NCH = _IPW * _CPI
_NBUF = 3
_STAG = 2
_ZR = 32
_NZ = (_IPW * _H) // _ZR


def _sc_body(in_hbm, keep_hbm, out_hbm, keep_v, zbuf, bufs, rsem, wsem, zsem):
    wid = lax.axis_index("s") * 2 + lax.axis_index("c")
    img0 = wid * _IPW

    pltpu.sync_copy(keep_hbm, keep_v)

    zv = jnp.zeros((16,), jnp.float32)
    nseg = _W // 16

    def _zinit(i, carry):
        r = i // nseg
        k = i - r * nseg
        zbuf[r, pl.ds(k * 16, 16)] = zv
        return carry

    lax.fori_loop(0, _ZR * nseg, _zinit, 0)

    kvec = keep_v[...]

    for c in range(_C):
        keep_c = kvec[c]

        def _in_chunk(i, c=c):
            img, r = divmod(i, _CPI)
            return in_hbm.at[c, img0 + img, pl.ds(r * _CR, _CR)]

        def _out_chunk(i, c=c):
            img, r = divmod(i, _CPI)
            return out_hbm.at[c, img0 + img, pl.ds(r * _CR, _CR)]

        @pl.when(keep_c > 0)
        def _copy(c=c, _in_chunk=_in_chunk, _out_chunk=_out_chunk):
            def _read(i):
                pltpu.make_async_copy(
                    _in_chunk(i), bufs.at[i % _NBUF], rsem.at[i % _NBUF]
                ).start()

            def _wait_read(i):
                pltpu.make_async_copy(
                    _in_chunk(i), bufs.at[i % _NBUF], rsem.at[i % _NBUF]
                ).wait()

            def _write(i):
                pltpu.make_async_copy(
                    bufs.at[i % _NBUF], _out_chunk(i), wsem.at[i % _NBUF]
                ).start()

            def _wait_write(i):
                pltpu.make_async_copy(
                    bufs.at[i % _NBUF], _out_chunk(i), wsem.at[i % _NBUF]
                ).wait()

            for i in range(_NCH + _STAG):
                if i < _NCH:
                    if i >= _NBUF:
                        _wait_write(i - _NBUF)
                    _read(i)
                j = i - _STAG
                if 0 <= j < _NCH:
                    _wait_read(j)
                    _write(j)
            for j in range(_NCH - _NBUF, _NCH):
                _wait_write(j)

        @pl.when(keep_c == 0)
        def _zero_out(c=c):
            def _zchunk(i, c=c):
                img, r = divmod(i, _H // _ZR)
                return out_hbm.at[c, img0 + img, pl.ds(r * _ZR, _ZR)]

            for i in range(_NZ):
                pltpu.make_async_copy(zbuf, _zchunk(i), zsem).start()
            for i in range(_NZ):
                pltpu.make_async_copy(zbuf, _zchunk(i), zsem).wait()


@functools.partial(
    pl.kernel,
    mesh=plsc.VectorSubcoreMesh(core_axis_name="c", subcore_axis_name="s"),
    out_type=jax.ShapeDtypeStruct((_C, _IMG, _H, _W), jnp.float32),
    scratch_types=[
        pltpu.VMEM((16,), jnp.int32),
        pltpu.VMEM((_ZR, _W), jnp.float32),
        pltpu.VMEM((_NBUF, _CR, _W), jnp.float32),
        pltpu.SemaphoreType.DMA((_NBUF,)),
        pltpu.SemaphoreType.DMA((_NBUF,)),
        pltpu.SemaphoreType.DMA,
    ],
)
def _sc_kernel(in_hbm, keep_hbm, out_hbm, keep_v, zbuf, bufs, rsem, wsem, zsem):
    _sc_body(in_hbm, keep_hbm, out_hbm, keep_v, zbuf, bufs, rsem, wsem, zsem)


def kernel(tensor, skip_prob):
    u = jax.random.uniform(jax.random.key(42), (3,), dtype=jnp.float32)
    keep = (u > skip_prob).astype(jnp.int32)
    keep16 = jnp.pad(keep, (0, 16 - _C))
    return _sc_kernel(tensor, keep16)


# speedup vs baseline: 1.3134x; 1.3134x over previous
import jax
import jax.numpy as jnp
from jax.experimental import pallas as pl
from jax.experimental.pallas import tpu as pltpu

_C = 3
_IMG = 64
_H = 512
_W = 512
_IPC = 4
_CPC = _IMG // _IPC
_NCHUNKS = _C * _CPC
_NBUF = 12
_D = 6


def _body(keep_ref, in_hbm, out_hbm, bufs, zbuf, rsem, wsem):
    zbuf[...] = jnp.zeros_like(zbuf)

    def in_chunk(i):
        c, r = divmod(i, _CPC)
        return in_hbm.at[c, pl.ds(r * _IPC, _IPC)]

    def out_chunk(i):
        c, r = divmod(i, _CPC)
        return out_hbm.at[c, pl.ds(r * _IPC, _IPC)]

    def start_read(i):
        b = i % _NBUF
        kc = keep_ref[i // _CPC]

        @pl.when(kc > 0)
        def _():
            pltpu.make_async_copy(in_chunk(i), bufs.at[b], rsem.at[b]).start()

    def start_write(p):
        b = p % _NBUF
        kc = keep_ref[p // _CPC]

        @pl.when(kc > 0)
        def _():
            pltpu.make_async_copy(in_chunk(p), bufs.at[b], rsem.at[b]).wait()
            pltpu.make_async_copy(bufs.at[b], out_chunk(p), wsem.at[b]).start()

        @pl.when(kc == 0)
        def _():
            pltpu.make_async_copy(zbuf, out_chunk(p), wsem.at[b]).start()

    for i in range(_NCHUNKS + _D):
        if i < _NCHUNKS:
            if i >= _NBUF:
                b = i % _NBUF
                pltpu.make_async_copy(
                    bufs.at[b], out_chunk(i - _NBUF), wsem.at[b]
                ).wait()
            start_read(i)
        if i >= _D:
            start_write(i - _D)

    for p in range(_NCHUNKS - _NBUF, _NCHUNKS):
        b = p % _NBUF
        pltpu.make_async_copy(bufs.at[b], out_chunk(p), wsem.at[b]).wait()


def kernel(tensor, skip_prob):
    u = jax.random.uniform(jax.random.key(42), (3,), dtype=jnp.float32)
    keep = (u > skip_prob).astype(jnp.int32)
    return pl.pallas_call(
        _body,
        in_specs=[
            pl.BlockSpec(memory_space=pltpu.SMEM),
            pl.BlockSpec(memory_space=pl.ANY),
        ],
        out_specs=pl.BlockSpec(memory_space=pl.ANY),
        out_shape=jax.ShapeDtypeStruct((_C, _IMG, _H, _W), jnp.float32),
        scratch_shapes=[
            pltpu.VMEM((_NBUF, _IPC, _H, _W), jnp.float32),
            pltpu.VMEM((_IPC, _H, _W), jnp.float32),
            pltpu.SemaphoreType.DMA((_NBUF,)),
            pltpu.SemaphoreType.DMA((_NBUF,)),
        ],
    )(keep, tensor)
